# baseline (speedup 1.0000x reference)
import functools

import jax
import jax.numpy as jnp
from jax import lax
from jax.experimental import pallas as pl
from jax.experimental.pallas import tpu as pltpu
from jax.experimental.pallas import tpu_sc as plsc

_BT = 512
_SW = 8
_NW = 24


def _sc_lookup(region_table, region_ids, state_ids, nlcd_ids):
    B = region_ids.shape[0]
    info = plsc.get_sparse_core_info()
    nw = info.num_cores * info.num_subcores
    bw = B // nw
    mesh = plsc.VectorSubcoreMesh(core_axis_name="c", subcore_axis_name="s")

    @functools.partial(
        pl.kernel,
        mesh=mesh,
        out_type=(
            jax.ShapeDtypeStruct((B, 64), jnp.float32),
            jax.ShapeDtypeStruct((B * _SW,), jnp.float32),
            jax.ShapeDtypeStruct((B * _NW,), jnp.float32),
        ),
        scratch_types=[
            pltpu.VMEM((bw,), jnp.int32),
            pltpu.VMEM((bw,), jnp.int32),
            pltpu.VMEM((bw,), jnp.int32),
            pltpu.VMEM((bw, 64), jnp.float32),
            pltpu.VMEM((bw * _SW,), jnp.float32),
            pltpu.VMEM((bw * _NW,), jnp.float32),
            pltpu.SemaphoreType.DMA,
        ],
        compiler_params=pltpu.CompilerParams(use_tc_tiling_on_sc=False,
                                             needs_layout_passes=False),
    )
    def lookup_kernel(rtab, rids, sids, nids,
                      rout, sout, nout,
                      ridx_v, sidx_v, nidx_v, rrows, soh, noh, rsem):
        wid = lax.axis_index("s") * info.num_cores + lax.axis_index("c")
        base = wid * bw
        pltpu.sync_copy(rids.at[pl.ds(base, bw)], ridx_v)
        pltpu.sync_copy(sids.at[pl.ds(base, bw)], sidx_v)
        pltpu.sync_copy(nids.at[pl.ds(base, bw)], nidx_v)
        rcp = pltpu.async_copy(rtab.at[ridx_v], rrows, rsem)

        zeros = jnp.zeros((16,), jnp.float32)
        ones = jnp.ones((16,), jnp.float32)
        lane = lax.iota(jnp.int32, 16)

        def zero_body(j, _):
            soh[pl.ds(j * 16, 16)] = zeros
            return 0

        def zero_body_n(j, _):
            noh[pl.ds(j * 16, 16)] = zeros
            return 0

        lax.fori_loop(0, bw * _SW // 16, zero_body, 0)
        lax.fori_loop(0, bw * _NW // 16, zero_body_n, 0)

        def scat_s(j, _):
            ids = sidx_v[pl.ds(j * 16, 16)]
            flat = (j * 16 + lane) * _SW + ids
            plsc.store_scatter(soh, [flat], ones)
            return 0

        def scat_n(j, _):
            ids = nidx_v[pl.ds(j * 16, 16)]
            flat = (j * 16 + lane) * _NW + ids
            plsc.store_scatter(noh, [flat], ones)
            return 0

        lax.fori_loop(0, bw // 16, scat_s, 0)
        lax.fori_loop(0, bw // 16, scat_n, 0)

        pltpu.sync_copy(soh, sout.at[pl.ds(base * _SW, bw * _SW)])
        pltpu.sync_copy(noh, nout.at[pl.ds(base * _NW, bw * _NW)])
        rcp.wait()
        pltpu.sync_copy(rrows, rout.at[pl.ds(base, bw)])

    reg, soh, noh = lookup_kernel(region_table, region_ids,
                                  state_ids, nlcd_ids)
    return reg, soh.reshape(B, _SW), noh.reshape(B, _NW)


def _ln(x, g, b, eps=1e-5):
    m = jnp.mean(x, axis=-1, keepdims=True)
    d = x - m
    v = jnp.mean(d * d, axis=-1, keepdims=True)
    return d * (lax.rsqrt(v + eps) * g) + b


def _ln_mm(x, g, b, J, eps=1e-5):
    m = jnp.dot(x, J, preferred_element_type=jnp.float32)
    d = x - m
    v = jnp.dot(d * d, J, preferred_element_type=jnp.float32)
    return d * (lax.rsqrt(v + eps) * g) + b


def _gelu(x):
    return 0.5 * x * (1.0 + lax.erf(x * 0.7071067811865476))


def _tc_body(sc_ref, tmp_ref, reg_ref, soh_ref, noh_ref,
             Ws_ref, bs_ref, gs_ref, bes_ref,
             Wt_ref, bt_ref, gt_ref, bet_ref,
             stab_ref, ntab_ref, Wc_ref, bc_ref, pos_ref,
             Wcb_ref, bcb_ref, go_ref, beo_ref, J_ref, out_ref):
    f32 = jnp.float32
    J = J_ref[...]

    tmp = tmp_ref[...]
    Wt = Wt_ref[...]
    bt = bt_ref[...]
    gt = gt_ref[...]
    bet = bet_ref[...]
    acc = _gelu(_ln_mm(jnp.dot(tmp[:, 0, :], Wt, preferred_element_type=f32)
                       + bt, gt, bet, J))
    for k in range(1, 14):
        acc = acc + _gelu(_ln_mm(
            jnp.dot(tmp[:, k, :], Wt,
                    preferred_element_type=f32) + bt, gt, bet, J))
    pos_mean = jnp.mean(pos_ref[...], axis=0, keepdims=True)
    pooled = acc * (1.0 / 14.0) + pos_mean

    s_emb = _gelu(_ln_mm(jnp.dot(sc_ref[...], Ws_ref[...],
                                 preferred_element_type=f32) + bs_ref[...],
                         gs_ref[...], bes_ref[...], J))

    Wc = Wc_ref[...]
    sproj = jnp.dot(stab_ref[...], Wc[64:96, :],
                    preferred_element_type=f32)
    nproj = jnp.dot(ntab_ref[...], Wc[96:128, :],
                    preferred_element_type=f32)
    cat = (jnp.dot(reg_ref[...], Wc[0:64, :], preferred_element_type=f32)
           + jnp.dot(soh_ref[:, 0:5], sproj, preferred_element_type=f32)
           + jnp.dot(noh_ref[:, 0:20], nproj, preferred_element_type=f32)
           + bc_ref[...])

    Wcb = Wcb_ref[...]
    out = (jnp.dot(s_emb, Wcb[0:128, :], preferred_element_type=f32)
           + jnp.dot(pooled, Wcb[128:256, :], preferred_element_type=f32)
           + jnp.dot(cat, Wcb[256:384, :], preferred_element_type=f32)
           + bcb_ref[...])
    out_ref[...] = _ln_mm(out, go_ref[...], beo_ref[...], J)


def _tc_fused(static_cont, temporal2d, reg_emb, soh, noh,
              W_static, b_static, g_static, be_static,
              W_temp, b_temp, g_temp, be_temp,
              state_table, nlcd_table, W_cat, b_cat, pos2d,
              W_combine, b_combine, g_out, be_out):
    B = static_cont.shape[0]
    grid = (B // _BT,)

    def row_spec(*dims):
        n = len(dims)
        return pl.BlockSpec((_BT,) + dims,
                            lambda i: (i,) + tuple(0 for _ in range(n)))

    def full_spec(shape):
        return pl.BlockSpec(shape, lambda i: tuple(0 for _ in shape))

    in_specs = [
        row_spec(50),
        row_spec(14, 20),
        row_spec(64),
        row_spec(_SW),
        row_spec(_NW),
        full_spec((50, 128)),
        full_spec((1, 128)),
        full_spec((1, 128)),
        full_spec((1, 128)),
        full_spec((20, 128)),
        full_spec((1, 128)),
        full_spec((1, 128)),
        full_spec((1, 128)),
        full_spec((5, 32)),
        full_spec((20, 32)),
        full_spec((128, 128)),
        full_spec((1, 128)),
        full_spec((14, 128)),
        full_spec((384, 128)),
        full_spec((1, 128)),
        full_spec((1, 128)),
        full_spec((1, 128)),
        full_spec((128, 128)),
    ]
    return pl.pallas_call(
        _tc_body,
        grid=grid,
        in_specs=in_specs,
        out_specs=pl.BlockSpec((_BT, 128), lambda i: (i, 0)),
        out_shape=jax.ShapeDtypeStruct((B, 128), jnp.float32),
    )(static_cont, temporal2d, reg_emb, soh, noh,
      W_static, b_static, g_static, be_static,
      W_temp, b_temp, g_temp, be_temp,
      state_table, nlcd_table, W_cat, b_cat, pos2d,
      W_combine, b_combine, g_out, be_out,
      jnp.full((128, 128), 1.0 / 128.0, jnp.float32))


def kernel(static_cont, temporal, region_ids, state_ids, nlcd_ids,
           W_static, b_static, g_static, be_static,
           W_temp, b_temp, g_temp, be_temp,
           region_table, state_table, nlcd_table,
           W_cat, b_cat, temporal_pos,
           W_combine, b_combine, g_out, be_out):
    reg_emb, soh, noh = _sc_lookup(
        region_table, region_ids.astype(jnp.int32),
        state_ids.astype(jnp.int32), nlcd_ids.astype(jnp.int32))
    B = static_cont.shape[0]
    row = lambda v: v.reshape(1, -1)
    return _tc_fused(
        static_cont, temporal, reg_emb, soh, noh,
        W_static, row(b_static), row(g_static), row(be_static),
        W_temp, row(b_temp), row(g_temp), row(be_temp),
        state_table, nlcd_table, W_cat, row(b_cat),
        temporal_pos.reshape(14, 128),
        W_combine, row(b_combine), row(g_out), row(be_out))


# speedup vs baseline: 1.1227x; 1.1227x over previous
import functools

import jax
import jax.numpy as jnp
from jax import lax
from jax.experimental import pallas as pl
from jax.experimental.pallas import tpu as pltpu
from jax.experimental.pallas import tpu_sc as plsc

_BT = 512
_SW = 8
_NW = 24


def _sc_lookup(region_table, region_ids, state_ids, nlcd_ids):
    B = region_ids.shape[0]
    info = plsc.get_sparse_core_info()
    nw = info.num_cores * info.num_subcores
    bw = B // nw
    mesh = plsc.VectorSubcoreMesh(core_axis_name="c", subcore_axis_name="s")

    @functools.partial(
        pl.kernel,
        mesh=mesh,
        out_type=(
            jax.ShapeDtypeStruct((B, 64), jnp.float32),
            jax.ShapeDtypeStruct((B * _SW,), jnp.float32),
            jax.ShapeDtypeStruct((B * _NW,), jnp.float32),
        ),
        scratch_types=[
            pltpu.VMEM((bw,), jnp.int32),
            pltpu.VMEM((bw,), jnp.int32),
            pltpu.VMEM((bw,), jnp.int32),
            pltpu.VMEM((bw, 64), jnp.float32),
            pltpu.VMEM((bw * _SW,), jnp.float32),
            pltpu.VMEM((bw * _NW,), jnp.float32),
            pltpu.SemaphoreType.DMA,
        ],
        compiler_params=pltpu.CompilerParams(use_tc_tiling_on_sc=False,
                                             needs_layout_passes=False),
    )
    def lookup_kernel(rtab, rids, sids, nids,
                      rout, sout, nout,
                      ridx_v, sidx_v, nidx_v, rrows, soh, noh, rsem):
        wid = lax.axis_index("s") * info.num_cores + lax.axis_index("c")
        base = wid * bw
        pltpu.sync_copy(rids.at[pl.ds(base, bw)], ridx_v)
        pltpu.sync_copy(sids.at[pl.ds(base, bw)], sidx_v)
        pltpu.sync_copy(nids.at[pl.ds(base, bw)], nidx_v)
        rcp = pltpu.async_copy(rtab.at[ridx_v], rrows, rsem)

        zeros = jnp.zeros((16,), jnp.float32)
        ones = jnp.ones((16,), jnp.float32)
        lane = lax.iota(jnp.int32, 16)

        def zero_body(j, _):
            soh[pl.ds(j * 16, 16)] = zeros
            return 0

        def zero_body_n(j, _):
            noh[pl.ds(j * 16, 16)] = zeros
            return 0

        lax.fori_loop(0, bw * _SW // 16, zero_body, 0)
        lax.fori_loop(0, bw * _NW // 16, zero_body_n, 0)

        def scat_s(j, _):
            ids = sidx_v[pl.ds(j * 16, 16)]
            flat = (j * 16 + lane) * _SW + ids
            plsc.store_scatter(soh, [flat], ones)
            return 0

        def scat_n(j, _):
            ids = nidx_v[pl.ds(j * 16, 16)]
            flat = (j * 16 + lane) * _NW + ids
            plsc.store_scatter(noh, [flat], ones)
            return 0

        lax.fori_loop(0, bw // 16, scat_s, 0)
        lax.fori_loop(0, bw // 16, scat_n, 0)

        pltpu.sync_copy(soh, sout.at[pl.ds(base * _SW, bw * _SW)])
        pltpu.sync_copy(noh, nout.at[pl.ds(base * _NW, bw * _NW)])
        rcp.wait()
        pltpu.sync_copy(rrows, rout.at[pl.ds(base, bw)])

    reg, soh, noh = lookup_kernel(region_table, region_ids,
                                  state_ids, nlcd_ids)
    return reg, soh.reshape(B, _SW), noh.reshape(B, _NW)


def _ln(x, g, b, eps=1e-5):
    m = jnp.mean(x, axis=-1, keepdims=True)
    d = x - m
    v = jnp.mean(d * d, axis=-1, keepdims=True)
    return d * (lax.rsqrt(v + eps) * g) + b


def _ln_mm(x, g, b, J, eps=1e-5):
    m = jnp.dot(x, J, preferred_element_type=jnp.float32)
    d = x - m
    v = jnp.dot(d * d, J, preferred_element_type=jnp.float32)
    return d * (lax.rsqrt(v + eps) * g) + b


def _gelu(x):
    return 0.5 * x * (1.0 + lax.erf(x * 0.7071067811865476))


def _tc_dense_body(sc_ref, tmp_ref,
                   Ws_ref, bs_ref, gs_ref, bes_ref,
                   Wt_ref, bt_ref, gt_ref, bet_ref,
                   pos_ref, Wcb_ref, bcb_ref, J_ref, out_ref):
    f32 = jnp.float32
    J = J_ref[...]

    tmp = tmp_ref[...]
    Wt = Wt_ref[...]
    bt = bt_ref[...]
    gt = gt_ref[...]
    bet = bet_ref[...]
    acc = _gelu(_ln_mm(jnp.dot(tmp[:, 0:20], Wt, preferred_element_type=f32)
                       + bt, gt, bet, J))
    for k in range(1, 14):
        acc = acc + _gelu(_ln_mm(
            jnp.dot(tmp[:, 20 * k:20 * (k + 1)], Wt,
                    preferred_element_type=f32) + bt, gt, bet, J))
    pos_mean = jnp.mean(pos_ref[...], axis=0, keepdims=True)
    pooled = acc * (1.0 / 14.0) + pos_mean

    s_emb = _gelu(_ln_mm(jnp.dot(sc_ref[...], Ws_ref[...],
                                 preferred_element_type=f32) + bs_ref[...],
                         gs_ref[...], bes_ref[...], J))

    Wcb = Wcb_ref[...]
    out_ref[...] = (jnp.dot(s_emb, Wcb[0:128, :], preferred_element_type=f32)
                    + jnp.dot(pooled, Wcb[128:256, :],
                              preferred_element_type=f32)
                    + bcb_ref[...])


def _tc_cat_body(part_ref, reg_ref, soh_ref, noh_ref,
                 stab_ref, ntab_ref, Wc_ref, bc_ref,
                 Wcb_ref, go_ref, beo_ref, J_ref, out_ref):
    f32 = jnp.float32
    J = J_ref[...]

    Wc = Wc_ref[...]
    sproj = jnp.dot(stab_ref[...], Wc[64:96, :],
                    preferred_element_type=f32)
    nproj = jnp.dot(ntab_ref[...], Wc[96:128, :],
                    preferred_element_type=f32)
    cat = (jnp.dot(reg_ref[...], Wc[0:64, :], preferred_element_type=f32)
           + jnp.dot(soh_ref[:, 0:5], sproj, preferred_element_type=f32)
           + jnp.dot(noh_ref[:, 0:20], nproj, preferred_element_type=f32)
           + bc_ref[...])

    out = part_ref[...] + jnp.dot(cat, Wcb_ref[...][256:384, :],
                                  preferred_element_type=f32)
    out_ref[...] = _ln_mm(out, go_ref[...], beo_ref[...], J)


def _row_spec(*dims):
    n = len(dims)
    return pl.BlockSpec((_BT,) + dims,
                        lambda i: (i,) + tuple(0 for _ in range(n)))


def _full_spec(shape):
    return pl.BlockSpec(shape, lambda i: tuple(0 for _ in shape))


def _tc_fused(static_cont, temporal2d, reg_emb, soh, noh,
              W_static, b_static, g_static, be_static,
              W_temp, b_temp, g_temp, be_temp,
              state_table, nlcd_table, W_cat, b_cat, pos2d,
              W_combine, b_combine, g_out, be_out):
    B = static_cont.shape[0]
    grid = (B // _BT,)
    J = jnp.full((128, 128), 1.0 / 128.0, jnp.float32)

    dense_specs = [
        _row_spec(50),
        _row_spec(280),
        _full_spec((50, 128)),
        _full_spec((1, 128)),
        _full_spec((1, 128)),
        _full_spec((1, 128)),
        _full_spec((20, 128)),
        _full_spec((1, 128)),
        _full_spec((1, 128)),
        _full_spec((1, 128)),
        _full_spec((14, 128)),
        _full_spec((384, 128)),
        _full_spec((1, 128)),
        _full_spec((128, 128)),
    ]
    partial = pl.pallas_call(
        _tc_dense_body,
        grid=grid,
        in_specs=dense_specs,
        out_specs=pl.BlockSpec((_BT, 128), lambda i: (i, 0)),
        out_shape=jax.ShapeDtypeStruct((B, 128), jnp.float32),
    )(static_cont, temporal2d,
      W_static, b_static, g_static, be_static,
      W_temp, b_temp, g_temp, be_temp,
      pos2d, W_combine, b_combine, J)

    cat_specs = [
        _row_spec(128),
        _row_spec(64),
        _row_spec(_SW),
        _row_spec(_NW),
        _full_spec((5, 32)),
        _full_spec((20, 32)),
        _full_spec((128, 128)),
        _full_spec((1, 128)),
        _full_spec((384, 128)),
        _full_spec((1, 128)),
        _full_spec((1, 128)),
        _full_spec((128, 128)),
    ]
    return pl.pallas_call(
        _tc_cat_body,
        grid=grid,
        in_specs=cat_specs,
        out_specs=pl.BlockSpec((_BT, 128), lambda i: (i, 0)),
        out_shape=jax.ShapeDtypeStruct((B, 128), jnp.float32),
    )(partial, reg_emb, soh, noh,
      state_table, nlcd_table, W_cat, b_cat,
      W_combine, g_out, be_out, J)


def kernel(static_cont, temporal, region_ids, state_ids, nlcd_ids,
           W_static, b_static, g_static, be_static,
           W_temp, b_temp, g_temp, be_temp,
           region_table, state_table, nlcd_table,
           W_cat, b_cat, temporal_pos,
           W_combine, b_combine, g_out, be_out):
    reg_emb, soh, noh = _sc_lookup(
        region_table, region_ids.astype(jnp.int32),
        state_ids.astype(jnp.int32), nlcd_ids.astype(jnp.int32))
    B = static_cont.shape[0]
    row = lambda v: v.reshape(1, -1)
    return _tc_fused(
        static_cont, temporal.reshape(B, 14 * 20), reg_emb, soh, noh,
        W_static, row(b_static), row(g_static), row(be_static),
        W_temp, row(b_temp), row(g_temp), row(be_temp),
        state_table, nlcd_table, W_cat, row(b_cat),
        temporal_pos.reshape(14, 128),
        W_combine, row(b_combine), row(g_out), row(be_out))


# speedup vs baseline: 1.2236x; 1.0899x over previous
import functools

import jax
import jax.numpy as jnp
from jax import lax
from jax.experimental import pallas as pl
from jax.experimental.pallas import tpu as pltpu
from jax.experimental.pallas import tpu_sc as plsc

_BT = 512
_SW = 8
_NW = 24


def _sc_lookup(region_table, region_ids, state_ids, nlcd_ids):
    B = region_ids.shape[0]
    info = plsc.get_sparse_core_info()
    nw = info.num_cores * info.num_subcores
    bw = B // nw
    mesh = plsc.VectorSubcoreMesh(core_axis_name="c", subcore_axis_name="s")

    @functools.partial(
        pl.kernel,
        mesh=mesh,
        out_type=(
            jax.ShapeDtypeStruct((B, 64), jnp.float32),
            jax.ShapeDtypeStruct((B * _SW,), jnp.float32),
            jax.ShapeDtypeStruct((B * _NW,), jnp.float32),
        ),
        scratch_types=[
            pltpu.VMEM((bw,), jnp.int32),
            pltpu.VMEM((bw,), jnp.int32),
            pltpu.VMEM((bw,), jnp.int32),
            pltpu.VMEM((bw, 64), jnp.float32),
            pltpu.VMEM((bw * _SW,), jnp.float32),
            pltpu.VMEM((bw * _NW,), jnp.float32),
            pltpu.SemaphoreType.DMA,
        ],
        compiler_params=pltpu.CompilerParams(use_tc_tiling_on_sc=False,
                                             needs_layout_passes=False),
    )
    def lookup_kernel(rtab, rids, sids, nids,
                      rout, sout, nout,
                      ridx_v, sidx_v, nidx_v, rrows, soh, noh, rsem):
        wid = lax.axis_index("s") * info.num_cores + lax.axis_index("c")
        base = wid * bw
        pltpu.sync_copy(rids.at[pl.ds(base, bw)], ridx_v)
        pltpu.sync_copy(sids.at[pl.ds(base, bw)], sidx_v)
        pltpu.sync_copy(nids.at[pl.ds(base, bw)], nidx_v)
        rcp = pltpu.async_copy(rtab.at[ridx_v], rrows, rsem)

        zeros = jnp.zeros((16,), jnp.float32)
        ones = jnp.ones((16,), jnp.float32)
        lane = lax.iota(jnp.int32, 16)

        def zero_body(j, _):
            soh[pl.ds(j * 16, 16)] = zeros
            return 0

        def zero_body_n(j, _):
            noh[pl.ds(j * 16, 16)] = zeros
            return 0

        lax.fori_loop(0, bw * _SW // 16, zero_body, 0)
        lax.fori_loop(0, bw * _NW // 16, zero_body_n, 0)

        def scat_s(j, _):
            ids = sidx_v[pl.ds(j * 16, 16)]
            flat = (j * 16 + lane) * _SW + ids
            plsc.store_scatter(soh, [flat], ones)
            return 0

        def scat_n(j, _):
            ids = nidx_v[pl.ds(j * 16, 16)]
            flat = (j * 16 + lane) * _NW + ids
            plsc.store_scatter(noh, [flat], ones)
            return 0

        lax.fori_loop(0, bw // 16, scat_s, 0)
        lax.fori_loop(0, bw // 16, scat_n, 0)

        pltpu.sync_copy(soh, sout.at[pl.ds(base * _SW, bw * _SW)])
        pltpu.sync_copy(noh, nout.at[pl.ds(base * _NW, bw * _NW)])
        rcp.wait()
        pltpu.sync_copy(rrows, rout.at[pl.ds(base, bw)])

    reg, soh, noh = lookup_kernel(region_table, region_ids,
                                  state_ids, nlcd_ids)
    return reg, soh.reshape(B, _SW), noh.reshape(B, _NW)


def _ln(x, g, b, eps=1e-5):
    m = jnp.mean(x, axis=-1, keepdims=True)
    d = x - m
    v = jnp.mean(d * d, axis=-1, keepdims=True)
    return d * (lax.rsqrt(v + eps) * g) + b


def _ln_mm(x, g, b, J, eps=1e-5):
    m = jnp.dot(x, J, preferred_element_type=jnp.float32)
    d = x - m
    v = jnp.dot(d * d, J, preferred_element_type=jnp.float32)
    return d * (lax.rsqrt(v + eps) * g) + b


def _gelu(x):
    return 0.5 * x * (1.0 + lax.erf(x * 0.7071067811865476))


def _tc_body(sc_ref, tmp_ref, reg_ref, soh_ref, noh_ref,
             Ws_ref, bs_ref, gs_ref, bes_ref,
             Wt_ref, bt_ref, gt_ref, bet_ref,
             stab_ref, ntab_ref, Wc_ref, bc_ref, pos_ref,
             Wcb_ref, bcb_ref, go_ref, beo_ref, J_ref, out_ref):
    f32 = jnp.float32
    J = J_ref[...]

    tmp = tmp_ref[...]
    Wt = Wt_ref[...]
    bt = bt_ref[...]
    gt = gt_ref[...]
    bet = bet_ref[...]
    acc = _gelu(_ln_mm(jnp.dot(tmp[:, 0:20], Wt, preferred_element_type=f32)
                       + bt, gt, bet, J))
    for k in range(1, 14):
        acc = acc + _gelu(_ln_mm(
            jnp.dot(tmp[:, 20 * k:20 * (k + 1)], Wt,
                    preferred_element_type=f32) + bt, gt, bet, J))
    pos_mean = jnp.mean(pos_ref[...], axis=0, keepdims=True)
    pooled = acc * (1.0 / 14.0) + pos_mean

    s_emb = _gelu(_ln_mm(jnp.dot(sc_ref[...], Ws_ref[...],
                                 preferred_element_type=f32) + bs_ref[...],
                         gs_ref[...], bes_ref[...], J))

    Wc = Wc_ref[...]
    sproj = jnp.dot(stab_ref[...], Wc[64:96, :],
                    preferred_element_type=f32)
    nproj = jnp.dot(ntab_ref[...], Wc[96:128, :],
                    preferred_element_type=f32)
    cat = (jnp.dot(reg_ref[...], Wc[0:64, :], preferred_element_type=f32)
           + jnp.dot(soh_ref[:, 0:5], sproj, preferred_element_type=f32)
           + jnp.dot(noh_ref[:, 0:20], nproj, preferred_element_type=f32)
           + bc_ref[...])

    Wcb = Wcb_ref[...]
    out = (jnp.dot(s_emb, Wcb[0:128, :], preferred_element_type=f32)
           + jnp.dot(pooled, Wcb[128:256, :], preferred_element_type=f32)
           + jnp.dot(cat, Wcb[256:384, :], preferred_element_type=f32)
           + bcb_ref[...])
    out_ref[...] = _ln_mm(out, go_ref[...], beo_ref[...], J)


def _row_spec(*dims):
    n = len(dims)
    return pl.BlockSpec((_BT,) + dims,
                        lambda i: (i,) + tuple(0 for _ in range(n)))


def _full_spec(shape):
    return pl.BlockSpec(shape, lambda i: tuple(0 for _ in shape))


def _tc_fused(static_cont, temporal2d, reg_emb, soh, noh,
              W_static, b_static, g_static, be_static,
              W_temp, b_temp, g_temp, be_temp,
              state_table, nlcd_table, W_cat, b_cat, pos2d,
              W_combine, b_combine, g_out, be_out):
    B = static_cont.shape[0]
    grid = (B // _BT,)
    J = jnp.full((128, 128), 1.0 / 128.0, jnp.float32)

    specs = [
        _row_spec(50),
        _row_spec(280),
        _row_spec(64),
        _row_spec(_SW),
        _row_spec(_NW),
        _full_spec((50, 128)),
        _full_spec((1, 128)),
        _full_spec((1, 128)),
        _full_spec((1, 128)),
        _full_spec((20, 128)),
        _full_spec((1, 128)),
        _full_spec((1, 128)),
        _full_spec((1, 128)),
        _full_spec((5, 32)),
        _full_spec((20, 32)),
        _full_spec((128, 128)),
        _full_spec((1, 128)),
        _full_spec((14, 128)),
        _full_spec((384, 128)),
        _full_spec((1, 128)),
        _full_spec((1, 128)),
        _full_spec((1, 128)),
        _full_spec((128, 128)),
    ]
    return pl.pallas_call(
        _tc_body,
        grid=grid,
        in_specs=specs,
        out_specs=pl.BlockSpec((_BT, 128), lambda i: (i, 0)),
        out_shape=jax.ShapeDtypeStruct((B, 128), jnp.float32),
    )(static_cont, temporal2d, reg_emb, soh, noh,
      W_static, b_static, g_static, be_static,
      W_temp, b_temp, g_temp, be_temp,
      state_table, nlcd_table, W_cat, b_cat, pos2d,
      W_combine, b_combine, g_out, be_out, J)


def kernel(static_cont, temporal, region_ids, state_ids, nlcd_ids,
           W_static, b_static, g_static, be_static,
           W_temp, b_temp, g_temp, be_temp,
           region_table, state_table, nlcd_table,
           W_cat, b_cat, temporal_pos,
           W_combine, b_combine, g_out, be_out):
    reg_emb, soh, noh = _sc_lookup(
        region_table, region_ids.astype(jnp.int32),
        state_ids.astype(jnp.int32), nlcd_ids.astype(jnp.int32))
    B = static_cont.shape[0]
    row = lambda v: v.reshape(1, -1)
    return _tc_fused(
        static_cont, temporal.reshape(B, 14 * 20), reg_emb, soh, noh,
        W_static, row(b_static), row(g_static), row(be_static),
        W_temp, row(b_temp), row(g_temp), row(be_temp),
        state_table, nlcd_table, W_cat, row(b_cat),
        temporal_pos.reshape(14, 128),
        W_combine, row(b_combine), row(g_out), row(be_out))


# speedup vs baseline: 1.3539x; 1.1065x over previous
import functools

import jax
import jax.numpy as jnp
from jax import lax
from jax.experimental import pallas as pl
from jax.experimental.pallas import tpu as pltpu
from jax.experimental.pallas import tpu_sc as plsc

_BT = 1024
_SW = 8
_NW = 24


def _sc_lookup(region_table, region_ids, state_ids, nlcd_ids):
    B = region_ids.shape[0]
    info = plsc.get_sparse_core_info()
    nw = info.num_cores * info.num_subcores
    bw = B // nw
    mesh = plsc.VectorSubcoreMesh(core_axis_name="c", subcore_axis_name="s")

    @functools.partial(
        pl.kernel,
        mesh=mesh,
        out_type=(
            jax.ShapeDtypeStruct((B, 64), jnp.float32),
            jax.ShapeDtypeStruct((B * _SW,), jnp.float32),
            jax.ShapeDtypeStruct((B * _NW,), jnp.float32),
        ),
        scratch_types=[
            pltpu.VMEM((bw,), jnp.int32),
            pltpu.VMEM((bw,), jnp.int32),
            pltpu.VMEM((bw,), jnp.int32),
            pltpu.VMEM((bw, 64), jnp.float32),
            pltpu.VMEM((bw * _SW,), jnp.float32),
            pltpu.VMEM((bw * _NW,), jnp.float32),
            pltpu.SemaphoreType.DMA,
        ],
        compiler_params=pltpu.CompilerParams(use_tc_tiling_on_sc=False,
                                             needs_layout_passes=False),
    )
    def lookup_kernel(rtab, rids, sids, nids,
                      rout, sout, nout,
                      ridx_v, sidx_v, nidx_v, rrows, soh, noh, rsem):
        wid = lax.axis_index("s") * info.num_cores + lax.axis_index("c")
        base = wid * bw
        pltpu.sync_copy(rids.at[pl.ds(base, bw)], ridx_v)
        pltpu.sync_copy(sids.at[pl.ds(base, bw)], sidx_v)
        pltpu.sync_copy(nids.at[pl.ds(base, bw)], nidx_v)
        rcp = pltpu.async_copy(rtab.at[ridx_v], rrows, rsem)

        zeros = jnp.zeros((16,), jnp.float32)
        ones = jnp.ones((16,), jnp.float32)
        lane = lax.iota(jnp.int32, 16)

        def zero_body(j, _):
            soh[pl.ds(j * 16, 16)] = zeros
            return 0

        def zero_body_n(j, _):
            noh[pl.ds(j * 16, 16)] = zeros
            return 0

        lax.fori_loop(0, bw * _SW // 16, zero_body, 0)
        lax.fori_loop(0, bw * _NW // 16, zero_body_n, 0)

        def scat_s(j, _):
            ids = sidx_v[pl.ds(j * 16, 16)]
            flat = (j * 16 + lane) * _SW + ids
            plsc.store_scatter(soh, [flat], ones)
            return 0

        def scat_n(j, _):
            ids = nidx_v[pl.ds(j * 16, 16)]
            flat = (j * 16 + lane) * _NW + ids
            plsc.store_scatter(noh, [flat], ones)
            return 0

        lax.fori_loop(0, bw // 16, scat_s, 0)
        lax.fori_loop(0, bw // 16, scat_n, 0)

        pltpu.sync_copy(soh, sout.at[pl.ds(base * _SW, bw * _SW)])
        pltpu.sync_copy(noh, nout.at[pl.ds(base * _NW, bw * _NW)])
        rcp.wait()
        pltpu.sync_copy(rrows, rout.at[pl.ds(base, bw)])

    reg, soh, noh = lookup_kernel(region_table, region_ids,
                                  state_ids, nlcd_ids)
    return reg, soh.reshape(B, _SW), noh.reshape(B, _NW)


def _ln(x, g, b, eps=1e-5):
    m = jnp.mean(x, axis=-1, keepdims=True)
    d = x - m
    v = jnp.mean(d * d, axis=-1, keepdims=True)
    return d * (lax.rsqrt(v + eps) * g) + b


def _ln_mm(x, g, b, J, eps=1e-5):
    m = jnp.dot(x, J, preferred_element_type=jnp.float32)
    d = x - m
    v = jnp.dot(d * d, J, preferred_element_type=jnp.float32)
    return d * (lax.rsqrt(v + eps) * g) + b


def _gelu(x):
    return 0.5 * x * (1.0 + lax.erf(x * 0.7071067811865476))


def _tc_body(sc_ref, tmp_ref, reg_ref, soh_ref, noh_ref,
             Ws_ref, bs_ref, gs_ref, bes_ref,
             Wt_ref, bt_ref, gt_ref, bet_ref,
             stab_ref, ntab_ref, Wc_ref, bc_ref, pos_ref,
             Wcb_ref, bcb_ref, go_ref, beo_ref, J_ref, out_ref):
    f32 = jnp.float32
    J = J_ref[...]

    tmp = tmp_ref[...]
    Wt = Wt_ref[...]
    bt = bt_ref[...]
    gt = gt_ref[...]
    bet = bet_ref[...]
    acc = _gelu(_ln_mm(jnp.dot(tmp[:, 0:20], Wt, preferred_element_type=f32)
                       + bt, gt, bet, J))
    for k in range(1, 14):
        acc = acc + _gelu(_ln_mm(
            jnp.dot(tmp[:, 20 * k:20 * (k + 1)], Wt,
                    preferred_element_type=f32) + bt, gt, bet, J))
    pos_mean = jnp.mean(pos_ref[...], axis=0, keepdims=True)
    pooled = acc * (1.0 / 14.0) + pos_mean

    s_emb = _gelu(_ln_mm(jnp.dot(sc_ref[...], Ws_ref[...],
                                 preferred_element_type=f32) + bs_ref[...],
                         gs_ref[...], bes_ref[...], J))

    Wc = Wc_ref[...]
    sproj = jnp.dot(stab_ref[...], Wc[64:96, :],
                    preferred_element_type=f32)
    nproj = jnp.dot(ntab_ref[...], Wc[96:128, :],
                    preferred_element_type=f32)
    cat = (jnp.dot(reg_ref[...], Wc[0:64, :], preferred_element_type=f32)
           + jnp.dot(soh_ref[:, 0:5], sproj, preferred_element_type=f32)
           + jnp.dot(noh_ref[:, 0:20], nproj, preferred_element_type=f32)
           + bc_ref[...])

    Wcb = Wcb_ref[...]
    out = (jnp.dot(s_emb, Wcb[0:128, :], preferred_element_type=f32)
           + jnp.dot(pooled, Wcb[128:256, :], preferred_element_type=f32)
           + jnp.dot(cat, Wcb[256:384, :], preferred_element_type=f32)
           + bcb_ref[...])
    out_ref[...] = _ln_mm(out, go_ref[...], beo_ref[...], J)


def _row_spec(*dims):
    n = len(dims)
    return pl.BlockSpec((_BT,) + dims,
                        lambda i: (i,) + tuple(0 for _ in range(n)))


def _full_spec(shape):
    return pl.BlockSpec(shape, lambda i: tuple(0 for _ in shape))


def _tc_fused(static_cont, temporal2d, reg_emb, soh, noh,
              W_static, b_static, g_static, be_static,
              W_temp, b_temp, g_temp, be_temp,
              state_table, nlcd_table, W_cat, b_cat, pos2d,
              W_combine, b_combine, g_out, be_out):
    B = static_cont.shape[0]
    grid = (B // _BT,)
    J = jnp.full((128, 128), 1.0 / 128.0, jnp.float32)

    specs = [
        _row_spec(50),
        _row_spec(280),
        _row_spec(64),
        _row_spec(_SW),
        _row_spec(_NW),
        _full_spec((50, 128)),
        _full_spec((1, 128)),
        _full_spec((1, 128)),
        _full_spec((1, 128)),
        _full_spec((20, 128)),
        _full_spec((1, 128)),
        _full_spec((1, 128)),
        _full_spec((1, 128)),
        _full_spec((5, 32)),
        _full_spec((20, 32)),
        _full_spec((128, 128)),
        _full_spec((1, 128)),
        _full_spec((14, 128)),
        _full_spec((384, 128)),
        _full_spec((1, 128)),
        _full_spec((1, 128)),
        _full_spec((1, 128)),
        _full_spec((128, 128)),
    ]
    return pl.pallas_call(
        _tc_body,
        grid=grid,
        in_specs=specs,
        out_specs=pl.BlockSpec((_BT, 128), lambda i: (i, 0)),
        out_shape=jax.ShapeDtypeStruct((B, 128), jnp.float32),
    )(static_cont, temporal2d, reg_emb, soh, noh,
      W_static, b_static, g_static, be_static,
      W_temp, b_temp, g_temp, be_temp,
      state_table, nlcd_table, W_cat, b_cat, pos2d,
      W_combine, b_combine, g_out, be_out, J)


def kernel(static_cont, temporal, region_ids, state_ids, nlcd_ids,
           W_static, b_static, g_static, be_static,
           W_temp, b_temp, g_temp, be_temp,
           region_table, state_table, nlcd_table,
           W_cat, b_cat, temporal_pos,
           W_combine, b_combine, g_out, be_out):
    reg_emb, soh, noh = _sc_lookup(
        region_table, region_ids.astype(jnp.int32),
        state_ids.astype(jnp.int32), nlcd_ids.astype(jnp.int32))
    B = static_cont.shape[0]
    row = lambda v: v.reshape(1, -1)
    return _tc_fused(
        static_cont, temporal.reshape(B, 14 * 20), reg_emb, soh, noh,
        W_static, row(b_static), row(g_static), row(be_static),
        W_temp, row(b_temp), row(g_temp), row(be_temp),
        state_table, nlcd_table, W_cat, row(b_cat),
        temporal_pos.reshape(14, 128),
        W_combine, row(b_combine), row(g_out), row(be_out))


# speedup vs baseline: 1.4154x; 1.0454x over previous
import functools

import jax
import jax.numpy as jnp
from jax import lax
from jax.experimental import pallas as pl
from jax.experimental.pallas import tpu as pltpu
from jax.experimental.pallas import tpu_sc as plsc

_BT = 2048
_SW = 8
_NW = 24


def _sc_lookup(region_table, region_ids, state_ids, nlcd_ids):
    B = region_ids.shape[0]
    info = plsc.get_sparse_core_info()
    nw = info.num_cores * info.num_subcores
    bw = B // nw
    mesh = plsc.VectorSubcoreMesh(core_axis_name="c", subcore_axis_name="s")

    @functools.partial(
        pl.kernel,
        mesh=mesh,
        out_type=(
            jax.ShapeDtypeStruct((B, 64), jnp.float32),
            jax.ShapeDtypeStruct((B * _SW,), jnp.float32),
            jax.ShapeDtypeStruct((B * _NW,), jnp.float32),
        ),
        scratch_types=[
            pltpu.VMEM((bw,), jnp.int32),
            pltpu.VMEM((bw,), jnp.int32),
            pltpu.VMEM((bw,), jnp.int32),
            pltpu.VMEM((bw, 64), jnp.float32),
            pltpu.VMEM((bw * _SW,), jnp.float32),
            pltpu.VMEM((bw * _NW,), jnp.float32),
            pltpu.SemaphoreType.DMA,
        ],
        compiler_params=pltpu.CompilerParams(use_tc_tiling_on_sc=False,
                                             needs_layout_passes=False),
    )
    def lookup_kernel(rtab, rids, sids, nids,
                      rout, sout, nout,
                      ridx_v, sidx_v, nidx_v, rrows, soh, noh, rsem):
        wid = lax.axis_index("s") * info.num_cores + lax.axis_index("c")
        base = wid * bw
        pltpu.sync_copy(rids.at[pl.ds(base, bw)], ridx_v)
        pltpu.sync_copy(sids.at[pl.ds(base, bw)], sidx_v)
        pltpu.sync_copy(nids.at[pl.ds(base, bw)], nidx_v)
        rcp = pltpu.async_copy(rtab.at[ridx_v], rrows, rsem)

        zeros = jnp.zeros((16,), jnp.float32)
        ones = jnp.ones((16,), jnp.float32)
        lane = lax.iota(jnp.int32, 16)

        def zero_body(j, _):
            soh[pl.ds(j * 16, 16)] = zeros
            return 0

        def zero_body_n(j, _):
            noh[pl.ds(j * 16, 16)] = zeros
            return 0

        lax.fori_loop(0, bw * _SW // 16, zero_body, 0)
        lax.fori_loop(0, bw * _NW // 16, zero_body_n, 0)

        def scat_s(j, _):
            ids = sidx_v[pl.ds(j * 16, 16)]
            flat = (j * 16 + lane) * _SW + ids
            plsc.store_scatter(soh, [flat], ones)
            return 0

        def scat_n(j, _):
            ids = nidx_v[pl.ds(j * 16, 16)]
            flat = (j * 16 + lane) * _NW + ids
            plsc.store_scatter(noh, [flat], ones)
            return 0

        lax.fori_loop(0, bw // 16, scat_s, 0)
        lax.fori_loop(0, bw // 16, scat_n, 0)

        pltpu.sync_copy(soh, sout.at[pl.ds(base * _SW, bw * _SW)])
        pltpu.sync_copy(noh, nout.at[pl.ds(base * _NW, bw * _NW)])
        rcp.wait()
        pltpu.sync_copy(rrows, rout.at[pl.ds(base, bw)])

    reg, soh, noh = lookup_kernel(region_table, region_ids,
                                  state_ids, nlcd_ids)
    return reg, soh.reshape(B, _SW), noh.reshape(B, _NW)


def _ln(x, g, b, eps=1e-5):
    m = jnp.mean(x, axis=-1, keepdims=True)
    d = x - m
    v = jnp.mean(d * d, axis=-1, keepdims=True)
    return d * (lax.rsqrt(v + eps) * g) + b


def _ln_mm(x, g, b, J, eps=1e-5):
    m = jnp.dot(x, J, preferred_element_type=jnp.float32)
    d = x - m
    v = jnp.dot(d * d, J, preferred_element_type=jnp.float32)
    return d * (lax.rsqrt(v + eps) * g) + b


def _gelu(x):
    return 0.5 * x * (1.0 + lax.erf(x * 0.7071067811865476))


def _tc_body(sc_ref, tmp_ref, reg_ref, soh_ref, noh_ref,
             Ws_ref, bs_ref, gs_ref, bes_ref,
             Wt_ref, bt_ref, gt_ref, bet_ref,
             stab_ref, ntab_ref, Wc_ref, bc_ref, pos_ref,
             Wcb_ref, bcb_ref, go_ref, beo_ref, J_ref, out_ref):
    f32 = jnp.float32
    J = J_ref[...]

    tmp = tmp_ref[...]
    Wt = Wt_ref[...]
    bt = bt_ref[...]
    gt = gt_ref[...]
    bet = bet_ref[...]
    acc = _gelu(_ln_mm(jnp.dot(tmp[:, 0:20], Wt, preferred_element_type=f32)
                       + bt, gt, bet, J))
    for k in range(1, 14):
        acc = acc + _gelu(_ln_mm(
            jnp.dot(tmp[:, 20 * k:20 * (k + 1)], Wt,
                    preferred_element_type=f32) + bt, gt, bet, J))
    pos_mean = jnp.mean(pos_ref[...], axis=0, keepdims=True)
    pooled = acc * (1.0 / 14.0) + pos_mean

    s_emb = _gelu(_ln_mm(jnp.dot(sc_ref[...], Ws_ref[...],
                                 preferred_element_type=f32) + bs_ref[...],
                         gs_ref[...], bes_ref[...], J))

    Wc = Wc_ref[...]
    sproj = jnp.dot(stab_ref[...], Wc[64:96, :],
                    preferred_element_type=f32)
    nproj = jnp.dot(ntab_ref[...], Wc[96:128, :],
                    preferred_element_type=f32)
    cat = (jnp.dot(reg_ref[...], Wc[0:64, :], preferred_element_type=f32)
           + jnp.dot(soh_ref[:, 0:5], sproj, preferred_element_type=f32)
           + jnp.dot(noh_ref[:, 0:20], nproj, preferred_element_type=f32)
           + bc_ref[...])

    Wcb = Wcb_ref[...]
    out = (jnp.dot(s_emb, Wcb[0:128, :], preferred_element_type=f32)
           + jnp.dot(pooled, Wcb[128:256, :], preferred_element_type=f32)
           + jnp.dot(cat, Wcb[256:384, :], preferred_element_type=f32)
           + bcb_ref[...])
    out_ref[...] = _ln_mm(out, go_ref[...], beo_ref[...], J)


def _row_spec(*dims):
    n = len(dims)
    return pl.BlockSpec((_BT,) + dims,
                        lambda i: (i,) + tuple(0 for _ in range(n)))


def _full_spec(shape):
    return pl.BlockSpec(shape, lambda i: tuple(0 for _ in shape))


def _tc_fused(static_cont, temporal2d, reg_emb, soh, noh,
              W_static, b_static, g_static, be_static,
              W_temp, b_temp, g_temp, be_temp,
              state_table, nlcd_table, W_cat, b_cat, pos2d,
              W_combine, b_combine, g_out, be_out):
    B = static_cont.shape[0]
    grid = (B // _BT,)
    J = jnp.full((128, 128), 1.0 / 128.0, jnp.float32)

    specs = [
        _row_spec(50),
        _row_spec(280),
        _row_spec(64),
        _row_spec(_SW),
        _row_spec(_NW),
        _full_spec((50, 128)),
        _full_spec((1, 128)),
        _full_spec((1, 128)),
        _full_spec((1, 128)),
        _full_spec((20, 128)),
        _full_spec((1, 128)),
        _full_spec((1, 128)),
        _full_spec((1, 128)),
        _full_spec((5, 32)),
        _full_spec((20, 32)),
        _full_spec((128, 128)),
        _full_spec((1, 128)),
        _full_spec((14, 128)),
        _full_spec((384, 128)),
        _full_spec((1, 128)),
        _full_spec((1, 128)),
        _full_spec((1, 128)),
        _full_spec((128, 128)),
    ]
    return pl.pallas_call(
        _tc_body,
        grid=grid,
        in_specs=specs,
        out_specs=pl.BlockSpec((_BT, 128), lambda i: (i, 0)),
        out_shape=jax.ShapeDtypeStruct((B, 128), jnp.float32),
    )(static_cont, temporal2d, reg_emb, soh, noh,
      W_static, b_static, g_static, be_static,
      W_temp, b_temp, g_temp, be_temp,
      state_table, nlcd_table, W_cat, b_cat, pos2d,
      W_combine, b_combine, g_out, be_out, J)


def kernel(static_cont, temporal, region_ids, state_ids, nlcd_ids,
           W_static, b_static, g_static, be_static,
           W_temp, b_temp, g_temp, be_temp,
           region_table, state_table, nlcd_table,
           W_cat, b_cat, temporal_pos,
           W_combine, b_combine, g_out, be_out):
    reg_emb, soh, noh = _sc_lookup(
        region_table, region_ids.astype(jnp.int32),
        state_ids.astype(jnp.int32), nlcd_ids.astype(jnp.int32))
    B = static_cont.shape[0]
    row = lambda v: v.reshape(1, -1)
    return _tc_fused(
        static_cont, temporal.reshape(B, 14 * 20), reg_emb, soh, noh,
        W_static, row(b_static), row(g_static), row(be_static),
        W_temp, row(b_temp), row(g_temp), row(be_temp),
        state_table, nlcd_table, W_cat, row(b_cat),
        temporal_pos.reshape(14, 128),
        W_combine, row(b_combine), row(g_out), row(be_out))
